# baseline (speedup 1.0000x reference)
import functools

import jax
import jax.numpy as jnp
from jax import lax
from jax.experimental import pallas as pl
from jax.experimental.pallas import tpu as pltpu
from jax.experimental.pallas import tpu_sc as plsc

NRBF = 20
GPa = 160.21766208

_NW = 32
_CH = 512


@functools.lru_cache(maxsize=None)
def _make_sc_gather(M, D, K):
    per_w = K // _NW
    nch = per_w // _CH
    assert per_w % _CH == 0 and K % (8 * _NW) == 0 and D % 16 == 0
    mesh = plsc.VectorSubcoreMesh(core_axis_name="c", subcore_axis_name="s")

    @functools.partial(
        pl.kernel, mesh=mesh,
        out_type=jax.ShapeDtypeStruct((K, D), jnp.float32),
        compiler_params=pltpu.CompilerParams(use_tc_tiling_on_sc=False),
        scratch_types=[
            pltpu.VMEM((_CH,), jnp.int32),
            pltpu.VMEM((_CH, D), jnp.float32),
            pltpu.SemaphoreType.DMA,
        ])
    def k(table_hbm, idx_hbm, out_hbm, idx_v, rows_v, sem):
        wid = lax.axis_index("s") * 2 + lax.axis_index("c")
        base0 = wid * per_w

        def body(i, carry):
            base = base0 + i * _CH
            pltpu.sync_copy(idx_hbm.at[pl.ds(base, _CH)], idx_v)
            pltpu.async_copy(table_hbm.at[idx_v], rows_v, sem).wait()
            pltpu.sync_copy(rows_v, out_hbm.at[pl.ds(base, _CH)])
            return carry

        lax.fori_loop(0, nch, body, 0)

    return k


def _pad_idx(idx, K):
    return jnp.concatenate(
        [idx, jnp.zeros((K - idx.shape[0],), dtype=idx.dtype)])


def _sc_gather(table, idx, K):
    return _make_sc_gather(table.shape[0], table.shape[1], K)(table, idx)


def _silu_grad(x, s):
    return s * (1.0 + x * (1.0 - s))


def kernel(atom_pos, cell, pbc_offsets, atom_attr, edge_index,
           three_body_indices, num_three_body, num_bonds, num_triple_ij,
           num_atoms, num_graphs, batch, atom_embedding, rbf_w, w_gate,
           w_msg, w_three, w_out):
    N = atom_pos.shape[0]
    E = edge_index.shape[1]
    G = cell.shape[0]
    T = three_body_indices.shape[0]
    NpG = N // G
    EpG = E // G
    TpG = T // G

    src = edge_index[0]
    dst = edge_index[1]
    g_src = src // NpG
    g_dst = dst // NpG

    bias = (jnp.arange(T, dtype=jnp.int32) // TpG) * EpG
    tb0 = three_body_indices[:, 0] + bias
    tb1 = three_body_indices[:, 1] + bias

    h0 = atom_embedding[atom_attr[:, 0]]
    centers = jnp.linspace(0.0, 25.0, NRBF)

    KE = ((E + 8 * _NW * _CH - 1) // (8 * _NW * _CH)) * (8 * _NW * _CH)
    KE = max(KE, _NW * _CH)
    while (KE // _NW) % _CH != 0:
        KE += 8 * _NW
    K2 = 2 * KE
    src_p = _pad_idx(src, KE)
    dst_p = _pad_idx(dst, KE)
    srcdst_p = jnp.concatenate([src_p, dst_p])
    tb_p = jnp.concatenate([_pad_idx(tb0, KE), _pad_idx(tb1, KE)])

    def core_fwd(pos_s, cell_s):
        pos_t = jnp.pad(pos_s, ((0, 0), (0, 13)))
        pp = _sc_gather(pos_t, srcdst_p, K2)
        ps = pp[:E, :3]
        pd = pp[KE:KE + E, :3]
        cell_e = cell_s[g_src]
        shift = jnp.einsum('ei,eij->ej', pbc_offsets, cell_e)
        rij = pd - ps + shift
        dist = jnp.sqrt(jnp.sum(rij * rij, axis=-1) + 1e-8)
        unit = rij / dist[:, None]
        w_ij = jnp.exp(-dist / 5.0)
        rbf = jnp.exp(-0.5 * (dist[:, None] - centers[None, :]) ** 2)

        feat_t = jnp.concatenate(
            [unit, w_ij[:, None], jnp.zeros((E, 12), jnp.float32)], axis=1)
        ft = _sc_gather(feat_t, tb_p, K2)
        u0 = ft[:E, :3]
        u1 = ft[KE:KE + E, :3]
        w0 = ft[:E, 3]
        w1 = ft[KE:KE + E, 3]
        cos_t = jnp.sum(u0 * u1, axis=-1)
        tm = cos_t * w0 * w1

        e_feat = rbf @ rbf_w + tm[:, None] * w_three[None, :]
        gate = jax.nn.sigmoid(e_feat @ w_gate)

        hs0 = _sc_gather(h0, src_p, KE)[:E]
        msgA = (hs0 * gate) @ w_msg
        aggA = jax.ops.segment_sum(msgA, dst, num_segments=N)
        sA = jax.nn.sigmoid(aggA)
        h1 = h0 + aggA * sA

        hs1 = _sc_gather(h1, src_p, KE)[:E]
        msgB = (hs1 * gate) @ w_msg
        aggB = jax.ops.segment_sum(msgB, dst, num_segments=N)
        sB = jax.nn.sigmoid(aggB)
        h2 = h1 + aggB * sB

        atom_e = h2 @ w_out
        energies = jnp.sum(atom_e.reshape(G, NpG), axis=1)
        res = (dist, unit, w_ij, cos_t, u0, u1, w0, w1, gate, hs0, hs1,
               aggA, sA, aggB, sB, ps, pd)
        return energies, res

    def core_bwd(res, ct):
        (dist, unit, w_ij, cos_t, u0, u1, w0, w1, gate, hs0, hs1,
         aggA, sA, aggB, sB, ps, pd) = res
        ctb = jnp.repeat(ct, NpG)
        dh2 = ctb[:, None] * w_out[None, :]
        dB = _silu_grad(aggB, sB) * dh2
        DBd = _sc_gather(dB, dst_p, KE)[:E] @ w_msg.T
        dh1 = dh2 + jax.ops.segment_sum(gate * DBd, src, num_segments=N)
        dA = _silu_grad(aggA, sA) * dh1
        DAd = _sc_gather(dA, dst_p, KE)[:E] @ w_msg.T

        dgate = hs1 * DBd + hs0 * DAd
        dz = dgate * gate * (1.0 - gate)
        de_feat = dz @ w_gate.T
        dtm = jnp.sum(de_feat * w_three[None, :], axis=-1)
        drbf = de_feat @ rbf_w.T
        rbf = jnp.exp(-0.5 * (dist[:, None] - centers[None, :]) ** 2)
        ddist_rbf = jnp.sum(drbf * (-(dist[:, None] - centers[None, :])) * rbf,
                            axis=-1)

        dcos = dtm * w0 * w1
        dw_e = (jax.ops.segment_sum(dtm * cos_t * w1, tb0, num_segments=E)
                + jax.ops.segment_sum(dtm * cos_t * w0, tb1, num_segments=E))
        dunit = (jax.ops.segment_sum(dcos[:, None] * u1, tb0, num_segments=E)
                 + jax.ops.segment_sum(dcos[:, None] * u0, tb1, num_segments=E))
        ddist = ddist_rbf + dw_e * (-w_ij / 5.0)

        gr = ((dunit - unit * jnp.sum(unit * dunit, axis=-1, keepdims=True))
              / dist[:, None] + ddist[:, None] * unit)

        dpos_s = (jax.ops.segment_sum(gr, dst, num_segments=N)
                  - jax.ops.segment_sum(gr, src, num_segments=N))
        dcell_s = jax.ops.segment_sum(
            pbc_offsets[:, :, None] * gr[:, None, :], g_src, num_segments=G)
        return (dpos_s, dcell_s)

    @jax.custom_vjp
    def core(pos_s, cell_s):
        return core_fwd(pos_s, cell_s)[0]

    core.defvjp(core_fwd, core_bwd)

    eye = jnp.eye(3, dtype=cell.dtype)[None]

    def energies_fn(pos, strain):
        cell_s = cell @ (eye + strain)
        strain_aug = strain[batch]
        pos_s = jnp.einsum('bi,bij->bj', pos, eye + strain_aug)
        return core(pos_s, cell_s)

    strain0 = jnp.zeros_like(cell)
    energies, vjp_fn = jax.vjp(energies_fn, atom_pos, strain0)
    g_pos, g_strain = vjp_fn(jnp.ones_like(energies))
    forces = -g_pos
    volume = jnp.linalg.det(cell)
    stresses = g_strain / volume[:, None, None] / GPa
    return (energies, forces, stresses)


# speedup vs baseline: 1.0763x; 1.0763x over previous
import functools

import jax
import jax.numpy as jnp
from jax import lax
from jax.experimental import pallas as pl
from jax.experimental.pallas import tpu as pltpu
from jax.experimental.pallas import tpu_sc as plsc

NRBF = 20
GPa = 160.21766208

_NW = 32
_CH = 896


@functools.lru_cache(maxsize=None)
def _make_sc_gather(M, D, K):
    per_w = K // _NW
    nch = per_w // _CH
    assert per_w % _CH == 0 and K % (8 * _NW) == 0 and D % 16 == 0
    mesh = plsc.VectorSubcoreMesh(core_axis_name="c", subcore_axis_name="s")

    @functools.partial(
        pl.kernel, mesh=mesh,
        out_type=jax.ShapeDtypeStruct((K, D), jnp.float32),
        compiler_params=pltpu.CompilerParams(use_tc_tiling_on_sc=False),
        scratch_types=[
            pltpu.VMEM((_CH,), jnp.int32),
            pltpu.VMEM((_CH, D), jnp.float32),
            pltpu.SemaphoreType.DMA,
        ])
    def k(table_hbm, idx_hbm, out_hbm, idx_v, rows_v, sem):
        wid = lax.axis_index("s") * 2 + lax.axis_index("c")
        base0 = wid * per_w

        def body(i, carry):
            base = base0 + i * _CH
            pltpu.sync_copy(idx_hbm.at[pl.ds(base, _CH)], idx_v)
            pltpu.async_copy(table_hbm.at[idx_v], rows_v, sem).wait()
            pltpu.sync_copy(rows_v, out_hbm.at[pl.ds(base, _CH)])
            return carry

        lax.fori_loop(0, nch, body, 0)

    return k


def _pad_idx(idx, K):
    return jnp.concatenate(
        [idx, jnp.zeros((K - idx.shape[0],), dtype=idx.dtype)])


def _sc_gather(table, idx, K):
    return _make_sc_gather(table.shape[0], table.shape[1], K)(table, idx)


def _silu_grad(x, s):
    return s * (1.0 + x * (1.0 - s))


def kernel(atom_pos, cell, pbc_offsets, atom_attr, edge_index,
           three_body_indices, num_three_body, num_bonds, num_triple_ij,
           num_atoms, num_graphs, batch, atom_embedding, rbf_w, w_gate,
           w_msg, w_three, w_out):
    N = atom_pos.shape[0]
    E = edge_index.shape[1]
    G = cell.shape[0]
    T = three_body_indices.shape[0]
    NpG = N // G
    EpG = E // G
    TpG = T // G

    src = edge_index[0]
    dst = edge_index[1]
    g_src = src // NpG
    g_dst = dst // NpG

    bias = (jnp.arange(T, dtype=jnp.int32) // TpG) * EpG
    tb0 = three_body_indices[:, 0] + bias
    tb1 = three_body_indices[:, 1] + bias

    h0 = atom_embedding[atom_attr[:, 0]]
    centers = jnp.linspace(0.0, 25.0, NRBF)

    KE = ((E + _NW * _CH - 1) // (_NW * _CH)) * (_NW * _CH)
    K2 = 2 * KE
    src_p = _pad_idx(src, KE)
    dst_p = _pad_idx(dst, KE)
    srcdst_p = jnp.concatenate([src_p, dst_p])
    tb_p = jnp.concatenate([_pad_idx(tb0, KE), _pad_idx(tb1, KE)])

    def core_fwd(pos_s, cell_s):
        pos_t = jnp.pad(pos_s, ((0, 0), (0, 13)))
        pp = _sc_gather(pos_t, srcdst_p, K2)
        ps = pp[:E, :3]
        pd = pp[KE:KE + E, :3]
        ge = g_src[:, None, None]
        cell_e = jnp.where(
            ge == 0, cell_s[0],
            jnp.where(ge == 1, cell_s[1],
                      jnp.where(ge == 2, cell_s[2], cell_s[3])))
        shift = jnp.einsum('ei,eij->ej', pbc_offsets, cell_e)
        rij = pd - ps + shift
        dist = jnp.sqrt(jnp.sum(rij * rij, axis=-1) + 1e-8)
        unit = rij / dist[:, None]
        w_ij = jnp.exp(-dist / 5.0)
        rbf = jnp.exp(-0.5 * (dist[:, None] - centers[None, :]) ** 2)

        feat_t = jnp.concatenate(
            [unit, w_ij[:, None], jnp.zeros((E, 12), jnp.float32)], axis=1)
        ft = _sc_gather(feat_t, tb_p, K2)
        u0 = ft[:E, :3]
        u1 = ft[KE:KE + E, :3]
        w0 = ft[:E, 3]
        w1 = ft[KE:KE + E, 3]
        cos_t = jnp.sum(u0 * u1, axis=-1)
        tm = cos_t * w0 * w1

        e_feat = rbf @ rbf_w + tm[:, None] * w_three[None, :]
        gate = jax.nn.sigmoid(e_feat @ w_gate)

        hs0 = _sc_gather(h0, src_p, KE)[:E]
        msgA = (hs0 * gate) @ w_msg
        aggA = jax.ops.segment_sum(msgA, dst, num_segments=N)
        sA = jax.nn.sigmoid(aggA)
        h1 = h0 + aggA * sA

        hs1 = _sc_gather(h1, src_p, KE)[:E]
        msgB = (hs1 * gate) @ w_msg
        aggB = jax.ops.segment_sum(msgB, dst, num_segments=N)
        sB = jax.nn.sigmoid(aggB)
        h2 = h1 + aggB * sB

        atom_e = h2 @ w_out
        energies = jnp.sum(atom_e.reshape(G, NpG), axis=1)
        res = (dist, unit, w_ij, cos_t, u0, u1, w0, w1, gate, hs0, hs1,
               aggA, sA, aggB, sB, ps, pd)
        return energies, res

    def core_bwd(res, ct):
        (dist, unit, w_ij, cos_t, u0, u1, w0, w1, gate, hs0, hs1,
         aggA, sA, aggB, sB, ps, pd) = res
        ctb = jnp.repeat(ct, NpG)
        dh2 = ctb[:, None] * w_out[None, :]
        dB = _silu_grad(aggB, sB) * dh2
        DBd = _sc_gather(dB, dst_p, KE)[:E] @ w_msg.T
        dh1 = dh2 + jax.ops.segment_sum(gate * DBd, src, num_segments=N)
        dA = _silu_grad(aggA, sA) * dh1
        DAd = _sc_gather(dA, dst_p, KE)[:E] @ w_msg.T

        dgate = hs1 * DBd + hs0 * DAd
        dz = dgate * gate * (1.0 - gate)
        de_feat = dz @ w_gate.T
        dtm = jnp.sum(de_feat * w_three[None, :], axis=-1)
        drbf = de_feat @ rbf_w.T
        rbf = jnp.exp(-0.5 * (dist[:, None] - centers[None, :]) ** 2)
        ddist_rbf = jnp.sum(drbf * (-(dist[:, None] - centers[None, :])) * rbf,
                            axis=-1)

        dcos = dtm * w0 * w1
        dw_e = (jax.ops.segment_sum(dtm * cos_t * w1, tb0, num_segments=E)
                + jax.ops.segment_sum(dtm * cos_t * w0, tb1, num_segments=E))
        dunit = (jax.ops.segment_sum(dcos[:, None] * u1, tb0, num_segments=E)
                 + jax.ops.segment_sum(dcos[:, None] * u0, tb1, num_segments=E))
        ddist = ddist_rbf + dw_e * (-w_ij / 5.0)

        gr = ((dunit - unit * jnp.sum(unit * dunit, axis=-1, keepdims=True))
              / dist[:, None] + ddist[:, None] * unit)

        dpos_s = (jax.ops.segment_sum(gr, dst, num_segments=N)
                  - jax.ops.segment_sum(gr, src, num_segments=N))
        dcell_s = jax.ops.segment_sum(
            pbc_offsets[:, :, None] * gr[:, None, :], g_src, num_segments=G)
        return (dpos_s, dcell_s)

    @jax.custom_vjp
    def core(pos_s, cell_s):
        return core_fwd(pos_s, cell_s)[0]

    core.defvjp(core_fwd, core_bwd)

    eye = jnp.eye(3, dtype=cell.dtype)[None]

    def energies_fn(pos, strain):
        cell_s = cell @ (eye + strain)
        strain_aug = strain[batch]
        pos_s = jnp.einsum('bi,bij->bj', pos, eye + strain_aug)
        return core(pos_s, cell_s)

    strain0 = jnp.zeros_like(cell)
    energies, vjp_fn = jax.vjp(energies_fn, atom_pos, strain0)
    g_pos, g_strain = vjp_fn(jnp.ones_like(energies))
    forces = -g_pos
    volume = jnp.linalg.det(cell)
    stresses = g_strain / volume[:, None, None] / GPa
    return (energies, forces, stresses)


# speedup vs baseline: 4.0100x; 3.7256x over previous
import functools

import jax
import jax.numpy as jnp
from jax import lax
from jax.experimental import pallas as pl
from jax.experimental.pallas import tpu as pltpu
from jax.experimental.pallas import tpu_sc as plsc

NRBF = 20
GPa = 160.21766208

_NW = 32
_CH = 896


@functools.lru_cache(maxsize=None)
def _make_sc_gather(M, D, K):
    per_w = K // _NW
    nch = per_w // _CH
    assert per_w % _CH == 0 and K % (8 * _NW) == 0 and D % 16 == 0
    mesh = plsc.VectorSubcoreMesh(core_axis_name="c", subcore_axis_name="s")

    @functools.partial(
        pl.kernel, mesh=mesh,
        out_type=jax.ShapeDtypeStruct((K, D), jnp.float32),
        compiler_params=pltpu.CompilerParams(use_tc_tiling_on_sc=False),
        scratch_types=[
            pltpu.VMEM((_CH,), jnp.int32),
            pltpu.VMEM((_CH, D), jnp.float32),
            pltpu.SemaphoreType.DMA,
        ])
    def k(table_hbm, idx_hbm, out_hbm, idx_v, rows_v, sem):
        wid = lax.axis_index("s") * 2 + lax.axis_index("c")
        base0 = wid * per_w

        def body(i, carry):
            base = base0 + i * _CH
            pltpu.sync_copy(idx_hbm.at[pl.ds(base, _CH)], idx_v)
            pltpu.async_copy(table_hbm.at[idx_v], rows_v, sem).wait()
            pltpu.sync_copy(rows_v, out_hbm.at[pl.ds(base, _CH)])
            return carry

        lax.fori_loop(0, nch, body, 0)

    return k


def _pad_idx(idx, K):
    return jnp.concatenate(
        [idx, jnp.zeros((K - idx.shape[0],), dtype=idx.dtype)])


def _sc_gather(table, idx, K):
    return _make_sc_gather(table.shape[0], table.shape[1], K)(table, idx)


def _silu_grad(x, s):
    return s * (1.0 + x * (1.0 - s))


def kernel(atom_pos, cell, pbc_offsets, atom_attr, edge_index,
           three_body_indices, num_three_body, num_bonds, num_triple_ij,
           num_atoms, num_graphs, batch, atom_embedding, rbf_w, w_gate,
           w_msg, w_three, w_out):
    N = atom_pos.shape[0]
    E = edge_index.shape[1]
    G = cell.shape[0]
    T = three_body_indices.shape[0]
    NpG = N // G
    EpG = E // G
    TpG = T // G

    src = edge_index[0]
    dst = edge_index[1]
    g_src = src // NpG
    g_dst = dst // NpG

    bias = (jnp.arange(T, dtype=jnp.int32) // TpG) * EpG
    tb0 = three_body_indices[:, 0] + bias
    tb1 = three_body_indices[:, 1] + bias

    centers = jnp.linspace(0.0, 25.0, NRBF)

    KE = ((E + _NW * _CH - 1) // (_NW * _CH)) * (_NW * _CH)
    K2 = 2 * KE
    KN = ((N + _NW * _CH - 1) // (_NW * _CH)) * (_NW * _CH)
    attr_p = _pad_idx(atom_attr[:, 0].astype(jnp.int32), KN)
    h0 = _sc_gather(atom_embedding, attr_p, KN)[:N]
    src_p = _pad_idx(src, KE)
    dst_p = _pad_idx(dst, KE)
    srcdst_p = jnp.concatenate([src_p, dst_p])
    tb_p = jnp.concatenate([_pad_idx(tb0, KE), _pad_idx(tb1, KE)])

    def core_fwd(pos_s, cell_s):
        pos_t = jnp.pad(pos_s, ((0, 0), (0, 13)))
        pp = _sc_gather(pos_t, srcdst_p, K2)
        ps = pp[:E, :3]
        pd = pp[KE:KE + E, :3]
        ge = g_src[:, None, None]
        cell_e = jnp.where(
            ge == 0, cell_s[0],
            jnp.where(ge == 1, cell_s[1],
                      jnp.where(ge == 2, cell_s[2], cell_s[3])))
        shift = jnp.einsum('ei,eij->ej', pbc_offsets, cell_e)
        rij = pd - ps + shift
        dist = jnp.sqrt(jnp.sum(rij * rij, axis=-1) + 1e-8)
        unit = rij / dist[:, None]
        w_ij = jnp.exp(-dist / 5.0)
        rbf = jnp.exp(-0.5 * (dist[:, None] - centers[None, :]) ** 2)

        feat_t = jnp.concatenate(
            [unit, w_ij[:, None], jnp.zeros((E, 12), jnp.float32)], axis=1)
        ft = _sc_gather(feat_t, tb_p, K2)
        u0 = ft[:E, :3]
        u1 = ft[KE:KE + E, :3]
        w0 = ft[:E, 3]
        w1 = ft[KE:KE + E, 3]
        cos_t = jnp.sum(u0 * u1, axis=-1)
        tm = cos_t * w0 * w1

        e_feat = rbf @ rbf_w + tm[:, None] * w_three[None, :]
        gate = jax.nn.sigmoid(e_feat @ w_gate)

        hs0 = _sc_gather(h0, src_p, KE)[:E]
        msgA = (hs0 * gate) @ w_msg
        aggA = jax.ops.segment_sum(msgA, dst, num_segments=N)
        sA = jax.nn.sigmoid(aggA)
        h1 = h0 + aggA * sA

        hs1 = _sc_gather(h1, src_p, KE)[:E]
        msgB = (hs1 * gate) @ w_msg
        aggB = jax.ops.segment_sum(msgB, dst, num_segments=N)
        sB = jax.nn.sigmoid(aggB)
        h2 = h1 + aggB * sB

        atom_e = h2 @ w_out
        energies = jnp.sum(atom_e.reshape(G, NpG), axis=1)
        res = (dist, unit, w_ij, cos_t, u0, u1, w0, w1, gate, hs0, hs1,
               aggA, sA, aggB, sB, ps, pd)
        return energies, res

    def core_bwd(res, ct):
        (dist, unit, w_ij, cos_t, u0, u1, w0, w1, gate, hs0, hs1,
         aggA, sA, aggB, sB, ps, pd) = res
        ctb = jnp.repeat(ct, NpG)
        dh2 = ctb[:, None] * w_out[None, :]
        dB = _silu_grad(aggB, sB) * dh2
        DBd = _sc_gather(dB, dst_p, KE)[:E] @ w_msg.T
        dh1 = dh2 + jax.ops.segment_sum(gate * DBd, src, num_segments=N)
        dA = _silu_grad(aggA, sA) * dh1
        DAd = _sc_gather(dA, dst_p, KE)[:E] @ w_msg.T

        dgate = hs1 * DBd + hs0 * DAd
        dz = dgate * gate * (1.0 - gate)
        de_feat = dz @ w_gate.T
        dtm = jnp.sum(de_feat * w_three[None, :], axis=-1)
        drbf = de_feat @ rbf_w.T
        rbf = jnp.exp(-0.5 * (dist[:, None] - centers[None, :]) ** 2)
        ddist_rbf = jnp.sum(drbf * (-(dist[:, None] - centers[None, :])) * rbf,
                            axis=-1)

        dcos = dtm * w0 * w1
        dw_e = (jax.ops.segment_sum(dtm * cos_t * w1, tb0, num_segments=E)
                + jax.ops.segment_sum(dtm * cos_t * w0, tb1, num_segments=E))
        dunit = (jax.ops.segment_sum(dcos[:, None] * u1, tb0, num_segments=E)
                 + jax.ops.segment_sum(dcos[:, None] * u0, tb1, num_segments=E))
        ddist = ddist_rbf + dw_e * (-w_ij / 5.0)

        gr = ((dunit - unit * jnp.sum(unit * dunit, axis=-1, keepdims=True))
              / dist[:, None] + ddist[:, None] * unit)

        dpos_s = (jax.ops.segment_sum(gr, dst, num_segments=N)
                  - jax.ops.segment_sum(gr, src, num_segments=N))
        dcell_s = jnp.stack([
            jnp.sum((g_src == g).astype(jnp.float32)[:, None, None]
                    * pbc_offsets[:, :, None] * gr[:, None, :], axis=0)
            for g in range(G)])
        return (dpos_s, dcell_s)

    @jax.custom_vjp
    def core(pos_s, cell_s):
        return core_fwd(pos_s, cell_s)[0]

    core.defvjp(core_fwd, core_bwd)

    eye = jnp.eye(3, dtype=cell.dtype)[None]

    def energies_fn(pos, strain):
        cell_s = cell @ (eye + strain)
        strain_aug = strain[batch]
        pos_s = jnp.einsum('bi,bij->bj', pos, eye + strain_aug)
        return core(pos_s, cell_s)

    strain0 = jnp.zeros_like(cell)
    energies, vjp_fn = jax.vjp(energies_fn, atom_pos, strain0)
    g_pos, g_strain = vjp_fn(jnp.ones_like(energies))
    forces = -g_pos
    volume = jnp.linalg.det(cell)
    stresses = g_strain / volume[:, None, None] / GPa
    return (energies, forces, stresses)
